# baseline (speedup 1.0000x reference)
import functools

import jax
import jax.numpy as jnp
from jax import lax
from jax.experimental import pallas as pl
from jax.experimental.pallas import tpu as pltpu
from jax.experimental.pallas import tpu_sc as plsc

EMBED = 64
LABELS = 4
B = 4096
L = 50

NC, NS, LANES = 2, 16, 16
NW = NC * NS
BPW = B // NW
LP = 64
IPW = BPW * LP
CHUNKS = IPW // 128
SEGS = EMBED // LANES
HB = B // 2

_mesh = plsc.VectorSubcoreMesh(
    core_axis_name="c", subcore_axis_name="s", num_cores=NC, num_subcores=NS
)


def _process_chunk(rows_v, slot, c, out_v):
    for e2 in range(2):
        def red(l, acc):
            r = e2 * LP + l
            return tuple(
                acc[g] + rows_v[slot, r, pl.ds(g * LANES, LANES)]
                for g in range(SEGS)
            )
        init = tuple(jnp.zeros((LANES,), jnp.float32) for _ in range(SEGS))
        acc = lax.fori_loop(0, L, red, init, unroll=5)
        for g in range(SEGS):
            out_v[2 * c + e2, pl.ds(g * LANES, LANES)] = acc[g] * (1.0 / L)


@functools.partial(
    pl.kernel,
    out_type=jax.ShapeDtypeStruct((HB, 2 * EMBED), jnp.float32),
    mesh=_mesh,
    scratch_types=[
        pltpu.VMEM((IPW,), jnp.int32),
        pltpu.VMEM((4, 128, EMBED), jnp.float32),
        pltpu.VMEM((BPW, EMBED), jnp.float32),
        pltpu.SemaphoreType.DMA,
        pltpu.SemaphoreType.DMA,
        pltpu.SemaphoreType.DMA,
        pltpu.SemaphoreType.DMA,
    ],
    compiler_params=pltpu.CompilerParams(use_tc_tiling_on_sc=False),
)
def _embed_bag(text_hbm, table_hbm, out_hbm, idx_v, rows_v, out_v,
               sem0, sem1, sem2, sem3):
    wid = lax.axis_index("s") * NC + lax.axis_index("c")
    sems = (sem0, sem1, sem2, sem3)
    pltpu.sync_copy(text_hbm.at[pl.ds(wid * IPW, IPW)], idx_v)

    def gather(j, s):
        return pltpu.async_copy(
            table_hbm.at[idx_v.at[pl.ds(j * 128, 128)]], rows_v.at[s], sems[s]
        )

    for s in range(4):
        gather(s, s)

    def quad_body(q, _):
        for s in range(4):
            j = 4 * q + s

            @pl.when(j < CHUNKS)
            def _():
                pltpu.make_async_copy(
                    table_hbm.at[idx_v.at[pl.ds(j * 128, 128)]],
                    rows_v.at[s], sems[s],
                ).wait()
                _process_chunk(rows_v, s, j, out_v)

                @pl.when(j + 4 < CHUNKS)
                def _():
                    gather(j + 4, s)
        return 0

    lax.fori_loop(0, (CHUNKS + 3) // 4, quad_body, 0)

    row_base = (wid & 15) * BPW
    col_base = (wid >> 4) * EMBED
    pltpu.sync_copy(
        out_v,
        out_hbm.at[pl.ds(row_base, BPW), pl.ds(col_base, EMBED)],
    )


VCB = 8192
VSB = 62
VHALF = VSB * VCB
VOCAB = 1000000


def _tpose_body(lo_ref, hi_ref, i_ref, o_ref):
    dn = (((0,), (0,)), ((), ()))
    ident = i_ref[...]
    o_ref[:, :EMBED] = lax.dot_general(
        lo_ref[...], ident, dn, preferred_element_type=jnp.float32)
    o_ref[:, EMBED:] = lax.dot_general(
        hi_ref[...], ident, dn, preferred_element_type=jnp.float32)


_repack_table = pl.pallas_call(
    _tpose_body,
    grid=(VSB,),
    in_specs=[
        pl.BlockSpec((EMBED, VCB), lambda i: (0, i)),
        pl.BlockSpec((EMBED, VCB), lambda i: (0, jnp.minimum(VSB + i, VOCAB // VCB))),
        pl.BlockSpec((EMBED, EMBED), lambda i: (0, 0)),
    ],
    out_specs=pl.BlockSpec((VCB, 2 * EMBED), lambda i: (i, 0)),
    out_shape=jax.ShapeDtypeStruct((VHALF, 2 * EMBED), jnp.float32),
    compiler_params=pltpu.CompilerParams(fuse_transposed_lhs_in_matmul=True),
)


def _head_body(emb_ref, w_ref, b_ref, out_ref):
    e = emb_ref[...]
    w = w_ref[...]
    bb = b_ref[...]
    dn = (((1,), (1,)), ((), ()))
    top = lax.dot_general(e[:, :EMBED], w, dn, preferred_element_type=jnp.float32)
    bot = lax.dot_general(e[:, EMBED:], w, dn, preferred_element_type=jnp.float32)
    out_ref[:HB, :] = 1.0 / (1.0 + jnp.exp(-(top + bb)))
    out_ref[HB:, :] = 1.0 / (1.0 + jnp.exp(-(bot + bb)))


_head = pl.pallas_call(
    _head_body,
    out_shape=jax.ShapeDtypeStruct((B, LABELS), jnp.float32),
)


def kernel(text, table, W, b):
    textp = jnp.concatenate([text, text[:, :LP - L]], axis=1)
    text1d = textp.reshape(B * LP)
    text1d = text1d * 2 - jnp.where(text1d >= VHALF, 2 * VHALF - 1, 0)
    tt = table.T
    eye = jnp.eye(EMBED, dtype=jnp.float32)
    table_rm = _repack_table(tt, tt, eye)
    table_rm = table_rm.reshape(2 * VHALF, EMBED)
    emb2 = _embed_bag(text1d, table_rm)
    return _head(emb2, W, b.reshape(1, LABELS))


# speedup vs baseline: 1.0604x; 1.0604x over previous
import functools

import jax
import jax.numpy as jnp
from jax import lax
from jax.experimental import pallas as pl
from jax.experimental.pallas import tpu as pltpu
from jax.experimental.pallas import tpu_sc as plsc

EMBED = 64
LABELS = 4
B = 4096
L = 50

NC, NS, LANES = 2, 16, 16
NW = NC * NS
BPW = B // NW
LP = 64
IPW = BPW * LP
CHUNKS = IPW // 128
SEGS = EMBED // LANES
HB = B // 2

_mesh = plsc.VectorSubcoreMesh(
    core_axis_name="c", subcore_axis_name="s", num_cores=NC, num_subcores=NS
)


def _process_chunk(rows_v, slot, c, out_v):
    for e2 in range(2):
        def red(l, acc):
            r = e2 * LP + l
            return tuple(
                acc[g] + rows_v[slot, r, pl.ds(g * LANES, LANES)]
                for g in range(SEGS)
            )
        init = tuple(jnp.zeros((LANES,), jnp.float32) for _ in range(SEGS))
        acc = lax.fori_loop(0, L, red, init, unroll=5)
        for g in range(SEGS):
            out_v[2 * c + e2, pl.ds(g * LANES, LANES)] = acc[g] * (1.0 / L)


@functools.partial(
    pl.kernel,
    out_type=jax.ShapeDtypeStruct((HB, 2 * EMBED), jnp.float32),
    mesh=_mesh,
    scratch_types=[
        pltpu.VMEM((IPW,), jnp.int32),
        pltpu.VMEM((4, 128, EMBED), jnp.float32),
        pltpu.VMEM((BPW, EMBED), jnp.float32),
        pltpu.SemaphoreType.DMA,
        pltpu.SemaphoreType.DMA,
        pltpu.SemaphoreType.DMA,
        pltpu.SemaphoreType.DMA,
    ],
    compiler_params=pltpu.CompilerParams(use_tc_tiling_on_sc=False),
)
def _embed_bag(text_hbm, table_hbm, out_hbm, idx_v, rows_v, out_v,
               sem0, sem1, sem2, sem3):
    wid = lax.axis_index("s") * NC + lax.axis_index("c")
    sems = (sem0, sem1, sem2, sem3)
    pltpu.sync_copy(text_hbm.at[pl.ds(wid * IPW, IPW)], idx_v)

    def gather(j, s):
        return pltpu.async_copy(
            table_hbm.at[idx_v.at[pl.ds(j * 128, 128)]], rows_v.at[s], sems[s]
        )

    for s in range(4):
        gather(s, s)

    def quad_body(q, _):
        for s in range(4):
            j = 4 * q + s

            @pl.when(j < CHUNKS)
            def _():
                pltpu.make_async_copy(
                    table_hbm.at[idx_v.at[pl.ds(j * 128, 128)]],
                    rows_v.at[s], sems[s],
                ).wait()
                _process_chunk(rows_v, s, j, out_v)

                @pl.when(j + 4 < CHUNKS)
                def _():
                    gather(j + 4, s)
        return 0

    lax.fori_loop(0, (CHUNKS + 3) // 4, quad_body, 0)

    row_base = (wid & 15) * BPW
    col_base = (wid >> 4) * EMBED
    pltpu.sync_copy(
        out_v,
        out_hbm.at[pl.ds(row_base, BPW), pl.ds(col_base, EMBED)],
    )


VCB = 16384
VSB = 31
VHALF = VSB * VCB
VOCAB = 1000000


def _tpose_body(lo_ref, hi_ref, o_ref):
    o_ref[:, :EMBED] = jnp.transpose(lo_ref[...], (1, 0))
    o_ref[:, EMBED:] = jnp.transpose(hi_ref[...], (1, 0))


_repack_table = pl.pallas_call(
    _tpose_body,
    grid=(VSB,),
    in_specs=[
        pl.BlockSpec((EMBED, VCB), lambda i: (0, i)),
        pl.BlockSpec((EMBED, VCB), lambda i: (0, jnp.minimum(VSB + i, VOCAB // VCB))),
    ],
    out_specs=pl.BlockSpec((VCB, 2 * EMBED), lambda i: (i, 0)),
    out_shape=jax.ShapeDtypeStruct((VHALF, 2 * EMBED), jnp.float32),
)


def _head_body(emb_ref, w_ref, b_ref, out_ref):
    e = emb_ref[...]
    w = w_ref[...]
    bb = b_ref[...]
    dn = (((1,), (1,)), ((), ()))
    top = lax.dot_general(e[:, :EMBED], w, dn, preferred_element_type=jnp.float32)
    bot = lax.dot_general(e[:, EMBED:], w, dn, preferred_element_type=jnp.float32)
    out_ref[:HB, :] = 1.0 / (1.0 + jnp.exp(-(top + bb)))
    out_ref[HB:, :] = 1.0 / (1.0 + jnp.exp(-(bot + bb)))


_head = pl.pallas_call(
    _head_body,
    out_shape=jax.ShapeDtypeStruct((B, LABELS), jnp.float32),
)


def kernel(text, table, W, b):
    textp = jnp.concatenate([text, text[:, :LP - L]], axis=1)
    text1d = textp.reshape(B * LP)
    text1d = text1d * 2 - jnp.where(text1d >= VHALF, 2 * VHALF - 1, 0)
    tt = table.T
    table_rm = _repack_table(tt, tt)
    table_rm = table_rm.reshape(2 * VHALF, EMBED)
    emb2 = _embed_bag(text1d, table_rm)
    return _head(emb2, W, b.reshape(1, LABELS))
